# baseline (device time: 147338 ns/iter reference)
import jax
import jax.numpy as jnp
from jax import lax
from jax.experimental import pallas as pl
from jax.experimental.pallas import tpu as pltpu

C = 8


def kernel(x):
    m, n = x.shape
    half = m // 2
    rows = half // C

    def body(x_ref, out_ref, p1_send, p1_recv, p2_send, p2_recv, copy_sem):
        my_x = lax.axis_index("x")
        my_y = lax.axis_index("y")
        xnbr = (1 - my_x, my_y)
        ynbr = (my_x, 1 - my_y)

        barrier = pltpu.get_barrier_semaphore()
        for nbr in (xnbr, ynbr):
            pl.semaphore_signal(
                barrier, inc=1, device_id=nbr,
                device_id_type=pl.DeviceIdType.MESH,
            )
        pl.semaphore_wait(barrier, 2)

        p1 = []
        for c in range(C):
            off = my_y * half + c * rows
            rdma = pltpu.make_async_remote_copy(
                src_ref=x_ref.at[pl.ds(off, rows), :],
                dst_ref=out_ref.at[pl.ds(my_x * m + off, rows), :],
                send_sem=p1_send.at[c],
                recv_sem=p1_recv.at[c],
                device_id=xnbr,
                device_id_type=pl.DeviceIdType.MESH,
            )
            rdma.start()
            p1.append(rdma)

        local = pltpu.make_async_copy(
            x_ref, out_ref.at[pl.ds(my_x * m, m), :], copy_sem
        )
        local.start()

        p2 = []
        for c in range(C):
            p1[c].wait_recv()
            off = (1 - my_x) * m + my_y * half + c * rows
            rdma = pltpu.make_async_remote_copy(
                src_ref=out_ref.at[pl.ds(off, rows), :],
                dst_ref=out_ref.at[pl.ds(off, rows), :],
                send_sem=p2_send.at[c],
                recv_sem=p2_recv.at[c],
                device_id=ynbr,
                device_id_type=pl.DeviceIdType.MESH,
            )
            rdma.start()
            p2.append(rdma)

        for c in range(C):
            p1[c].wait_send()
            p2[c].wait()
        local.wait()

    return pl.pallas_call(
        body,
        out_shape=jax.ShapeDtypeStruct((2 * m, n), x.dtype),
        in_specs=[pl.BlockSpec(memory_space=pltpu.VMEM)],
        out_specs=pl.BlockSpec(memory_space=pltpu.VMEM),
        scratch_shapes=[
            pltpu.SemaphoreType.DMA((C,)),
            pltpu.SemaphoreType.DMA((C,)),
            pltpu.SemaphoreType.DMA((C,)),
            pltpu.SemaphoreType.DMA((C,)),
            pltpu.SemaphoreType.DMA,
        ],
        compiler_params=pltpu.CompilerParams(collective_id=0),
    )(x)


# device time: 131619 ns/iter; 1.1194x vs baseline; 1.1194x over previous
import jax
import jax.numpy as jnp
from jax import lax
from jax.experimental import pallas as pl
from jax.experimental.pallas import tpu as pltpu

C = 16


def kernel(x):
    m, n = x.shape
    half = m // 2
    rows = half // C

    def body(x_ref, out_ref, p1_send, p1_recv, p2_send, p2_recv, copy_sem):
        my_x = lax.axis_index("x")
        my_y = lax.axis_index("y")
        xnbr = (1 - my_x, my_y)
        ynbr = (my_x, 1 - my_y)

        barrier = pltpu.get_barrier_semaphore()
        for nbr in (xnbr, ynbr):
            pl.semaphore_signal(
                barrier, inc=1, device_id=nbr,
                device_id_type=pl.DeviceIdType.MESH,
            )
        pl.semaphore_wait(barrier, 2)

        p1 = []
        for c in range(C):
            off = my_y * half + c * rows
            rdma = pltpu.make_async_remote_copy(
                src_ref=x_ref.at[pl.ds(off, rows), :],
                dst_ref=out_ref.at[pl.ds(my_x * m + off, rows), :],
                send_sem=p1_send.at[c],
                recv_sem=p1_recv.at[c],
                device_id=xnbr,
                device_id_type=pl.DeviceIdType.MESH,
            )
            rdma.start()
            p1.append(rdma)

        local = pltpu.make_async_copy(
            x_ref, out_ref.at[pl.ds(my_x * m, m), :], copy_sem
        )
        local.start()

        p2 = []
        for c in range(C):
            p1[c].wait_recv()
            off = (1 - my_x) * m + my_y * half + c * rows
            rdma = pltpu.make_async_remote_copy(
                src_ref=out_ref.at[pl.ds(off, rows), :],
                dst_ref=out_ref.at[pl.ds(off, rows), :],
                send_sem=p2_send.at[c],
                recv_sem=p2_recv.at[c],
                device_id=ynbr,
                device_id_type=pl.DeviceIdType.MESH,
            )
            rdma.start()
            p2.append(rdma)

        for c in range(C):
            p1[c].wait_send()
            p2[c].wait()
        local.wait()

    return pl.pallas_call(
        body,
        out_shape=jax.ShapeDtypeStruct((2 * m, n), x.dtype),
        in_specs=[pl.BlockSpec(memory_space=pl.ANY)],
        out_specs=pl.BlockSpec(memory_space=pl.ANY),
        scratch_shapes=[
            pltpu.SemaphoreType.DMA((C,)),
            pltpu.SemaphoreType.DMA((C,)),
            pltpu.SemaphoreType.DMA((C,)),
            pltpu.SemaphoreType.DMA((C,)),
            pltpu.SemaphoreType.DMA,
        ],
        compiler_params=pltpu.CompilerParams(collective_id=0),
    )(x)


# device time: 130147 ns/iter; 1.1321x vs baseline; 1.0113x over previous
import jax
import jax.numpy as jnp
from jax import lax
from jax.experimental import pallas as pl
from jax.experimental.pallas import tpu as pltpu

C = 32


def kernel(x):
    m, n = x.shape
    half = m // 2
    rows = half // C

    def body(x_ref, out_ref, p1_send, p1_recv, p2_send, p2_recv, copy_sem):
        my_x = lax.axis_index("x")
        my_y = lax.axis_index("y")
        xnbr = (1 - my_x, my_y)
        ynbr = (my_x, 1 - my_y)

        barrier = pltpu.get_barrier_semaphore()
        for nbr in (xnbr, ynbr):
            pl.semaphore_signal(
                barrier, inc=1, device_id=nbr,
                device_id_type=pl.DeviceIdType.MESH,
            )
        pl.semaphore_wait(barrier, 2)

        p1 = []
        for c in range(C):
            off = my_y * half + c * rows
            rdma = pltpu.make_async_remote_copy(
                src_ref=x_ref.at[pl.ds(off, rows), :],
                dst_ref=out_ref.at[pl.ds(my_x * m + off, rows), :],
                send_sem=p1_send.at[c],
                recv_sem=p1_recv.at[c],
                device_id=xnbr,
                device_id_type=pl.DeviceIdType.MESH,
            )
            rdma.start()
            p1.append(rdma)

        local = pltpu.make_async_copy(
            x_ref, out_ref.at[pl.ds(my_x * m, m), :], copy_sem
        )
        local.start()

        p2 = []
        for c in range(C):
            p1[c].wait_recv()
            off = (1 - my_x) * m + my_y * half + c * rows
            rdma = pltpu.make_async_remote_copy(
                src_ref=out_ref.at[pl.ds(off, rows), :],
                dst_ref=out_ref.at[pl.ds(off, rows), :],
                send_sem=p2_send.at[c],
                recv_sem=p2_recv.at[c],
                device_id=ynbr,
                device_id_type=pl.DeviceIdType.MESH,
            )
            rdma.start()
            p2.append(rdma)

        for c in range(C):
            p1[c].wait_send()
            p2[c].wait()
        local.wait()

    return pl.pallas_call(
        body,
        out_shape=jax.ShapeDtypeStruct((2 * m, n), x.dtype),
        in_specs=[pl.BlockSpec(memory_space=pl.ANY)],
        out_specs=pl.BlockSpec(memory_space=pl.ANY),
        scratch_shapes=[
            pltpu.SemaphoreType.DMA((C,)),
            pltpu.SemaphoreType.DMA((C,)),
            pltpu.SemaphoreType.DMA((C,)),
            pltpu.SemaphoreType.DMA((C,)),
            pltpu.SemaphoreType.DMA,
        ],
        compiler_params=pltpu.CompilerParams(collective_id=0),
    )(x)
